# Initial kernel scaffold; baseline (speedup 1.0000x reference)
import jax
import jax.numpy as jnp
from jax.experimental import pallas as pl


def kernel(x, adjs, previous_indices, W1, b1, W2, b2, Wo, bo):
    raise NotImplementedError("write your pallas kernel here")


# speedup vs baseline: 3.0165x; 3.0165x over previous
import functools

import jax
import jax.numpy as jnp
from jax import lax
from jax.experimental import pallas as pl
from jax.experimental.pallas import tpu as pltpu
from jax.experimental.pallas import tpu_sc as plsc

N = 10000
D = 128
NHID = 128
NCLS = 40
E = 320000

NC, NS, L = 2, 16, 16
NW = NC * NS
PADN = 10240
CH = 128
EPAD = 327680
NCH = EPAD // NW // CH
SUP = 16
NSUP = NCH // SUP
PADP = 12288
PR_T = PADP // NW
SC_ROWS = PADN // NS

_mesh = plsc.VectorSubcoreMesh(
    core_axis_name="c", subcore_axis_name="s", num_cores=NC, num_subcores=NS)


DRD = 16384


def _sc_body(h_hbm, src_hbm, dst_hbm, prev_hbm, za_hbm, z1_hbm,
             prevrows_out, agg_out, deg_out,
             src_v, dst_v, pidx_v, ebuf_v, hist_v,
             agg_sh, sem):
  c = lax.axis_index("c")
  s = lax.axis_index("s")
  wid = s * NC + c
  ones16 = jnp.ones((L,), jnp.float32)

  pltpu.sync_copy(z1_hbm, hist_v)
  pltpu.sync_copy(za_hbm, agg_sh.at[pl.ds(s * SC_ROWS, SC_ROWS)])

  for j in range(PR_T // CH):
    pltpu.sync_copy(prev_hbm.at[pl.ds(wid * PR_T + j * CH, CH)], pidx_v)
    pltpu.async_copy(h_hbm.at[pidx_v], ebuf_v, sem).wait()
    pltpu.sync_copy(ebuf_v, prevrows_out.at[pl.ds(wid * PR_T + j * CH, CH)])
  plsc.subcore_barrier()

  def sup(g, carry):
    pltpu.sync_copy(src_hbm.at[pl.ds(wid * NCH + g * SUP, SUP)], src_v)
    pltpu.sync_copy(dst_hbm.at[pl.ds(wid * NCH + g * SUP, SUP)], dst_v)

    def chunk(j, cc):
      pltpu.async_copy(h_hbm.at[src_v.at[j]], ebuf_v, sem).wait()
      pltpu.sync_copy(ebuf_v, agg_sh.at[dst_v.at[j]], add=True)
      for k in range(CH // L):
        plsc.addupdate_scatter(hist_v, [dst_v[j, pl.ds(k * L, L)]], ones16)
      return cc

    return lax.fori_loop(0, SUP, chunk, carry)

  lax.fori_loop(0, NSUP, sup, 0)
  plsc.subcore_barrier()

  rows = pl.ds(s * SC_ROWS, SC_ROWS)
  pltpu.sync_copy(agg_sh.at[rows], agg_out.at[c, rows])
  pltpu.sync_copy(hist_v, deg_out.at[pl.ds(wid * DRD, DRD)])


_sc_layer = pl.kernel(
    _sc_body,
    out_type=(
        jax.ShapeDtypeStruct((PADP, D), jnp.float32),
        jax.ShapeDtypeStruct((NC, PADN, D), jnp.float32),
        jax.ShapeDtypeStruct((NW * DRD,), jnp.float32),
    ),
    mesh=_mesh,
    compiler_params=pltpu.CompilerParams(needs_layout_passes=False),
    scratch_types=[
        pltpu.VMEM((SUP, CH), jnp.int32),
        pltpu.VMEM((SUP, CH), jnp.int32),
        pltpu.VMEM((CH,), jnp.int32),
        pltpu.VMEM((CH, D), jnp.float32),
        pltpu.VMEM((DRD,), jnp.float32),
        pltpu.VMEM_SHARED((PADN, D), jnp.float32),
        pltpu.SemaphoreType.DMA,
    ],
)

BR = 512


def _mm_body(x_ref, w_ref, b_ref, o_ref):
  o_ref[...] = jnp.dot(x_ref[...], w_ref[...],
                       preferred_element_type=jnp.float32,
                       precision=lax.Precision.HIGHEST) + b_ref[...]


def _tc_matmul(x, w, b):
  return pl.pallas_call(
      _mm_body,
      grid=(PADN // BR,),
      in_specs=[
          pl.BlockSpec((BR, D), lambda i: (i, 0)),
          pl.BlockSpec((D, D), lambda i: (0, 0)),
          pl.BlockSpec((1, D), lambda i: (0, 0)),
      ],
      out_specs=pl.BlockSpec((BR, D), lambda i: (i, 0)),
      out_shape=jax.ShapeDtypeStruct((PADN, D), jnp.float32),
  )(x, w, b)


def _comb_body(prev_ref, aggA_ref, aggB_ref, dh_ref,
               wt_ref, wb_ref, b_ref, o_ref):
  deg = jnp.maximum(jnp.sum(dh_ref[...], axis=1, keepdims=True), 1.0)
  hn = jnp.maximum((aggA_ref[...] + aggB_ref[...]) / deg, 0.0)
  hp = jnp.maximum(prev_ref[...], 0.0)
  o_ref[...] = (
      jnp.dot(hp, wt_ref[...], preferred_element_type=jnp.float32,
              precision=lax.Precision.HIGHEST)
      + jnp.dot(hn, wb_ref[...], preferred_element_type=jnp.float32,
                precision=lax.Precision.HIGHEST)
      + b_ref[...])


def _tc_combine(prevrows, agg, dh, wt, wb, b):
  return pl.pallas_call(
      _comb_body,
      grid=(PADN // BR,),
      in_specs=[
          pl.BlockSpec((BR, D), lambda i: (i, 0)),
          pl.BlockSpec((BR, D), lambda i: (i, 0)),
          pl.BlockSpec((BR, D), lambda i: (i, 0)),
          pl.BlockSpec((BR, NW), lambda i: (i, 0)),
          pl.BlockSpec((D, D), lambda i: (0, 0)),
          pl.BlockSpec((D, D), lambda i: (0, 0)),
          pl.BlockSpec((1, D), lambda i: (0, 0)),
      ],
      out_specs=pl.BlockSpec((BR, D), lambda i: (i, 0)),
      out_shape=jax.ShapeDtypeStruct((PADN, D), jnp.float32),
  )(prevrows, agg[0], agg[1], dh, wt, wb, b)


def kernel(x, adjs, previous_indices, W1, b1, W2, b2, Wo, bo):
  f32 = jnp.float32
  xp = jnp.zeros((PADN, D), f32).at[:N].set(x)
  za = jnp.zeros((SC_ROWS, D), f32)
  z1 = jnp.zeros((DRD,), f32)

  pad_dst = (N + (jnp.arange(EPAD - E) % (PADN - N))).astype(jnp.int32)
  pad_src = jnp.zeros((EPAD - E,), jnp.int32)
  pad_prev = jnp.zeros((PADP - N,), jnp.int32)

  def prep(layer):
    src = jnp.concatenate([adjs[layer, 1].astype(jnp.int32), pad_src])
    dst = jnp.concatenate([adjs[layer, 0].astype(jnp.int32), pad_dst])
    prev = jnp.concatenate(
        [previous_indices[layer].astype(jnp.int32), pad_prev])
    return src.reshape(-1, CH), dst.reshape(-1, CH), prev

  h1 = _tc_matmul(xp, W1, b1.reshape(1, D))

  src0, dst0, prev0 = prep(0)
  pr1, agg1, deg1 = _sc_layer(h1, src0, dst0, prev0, za, z1)
  dh1 = deg1.reshape(NW, DRD).T
  h2 = _tc_combine(pr1, agg1, dh1, W2[:NHID], W2[NHID:], b2.reshape(1, D))

  src1, dst1, prev1 = prep(1)
  pr2, agg2, deg2 = _sc_layer(h2, src1, dst1, prev1, za, z1)
  dh2 = deg2.reshape(NW, DRD).T
  wo_p = jnp.zeros((2 * NHID, D), f32).at[:, :NCLS].set(Wo)
  bo_p = jnp.zeros((1, D), f32).at[0, :NCLS].set(bo)
  outp = _tc_combine(pr2, agg2, dh2, wo_p[:NHID], wo_p[NHID:], bo_p)

  return outp[:N, :NCLS]


# speedup vs baseline: 6.2755x; 2.0804x over previous
import jax
import jax.numpy as jnp
from jax import lax
from jax.experimental import pallas as pl
from jax.experimental.pallas import tpu as pltpu
from jax.experimental.pallas import tpu_sc as plsc

N = 10000
D = 128
NHID = 128
NCLS = 40
E = 320000

NC, NS, L = 2, 16, 16
NW = NC * NS
PADN = 10112
CH = 128
EPAD = 327680
NCH = EPAD // NW // CH
SUP = 8
NSUP = NCH // SUP
PADP = 12288
PR_T = PADP // NW
SC_ROWS = PADN // NS

_mesh = plsc.VectorSubcoreMesh(
    core_axis_name="c", subcore_axis_name="s", num_cores=NC, num_subcores=NS)


def _sc_body(h_hbm, src_hbm, dst_hbm, prev_hbm, za_hbm, z1_hbm,
             prevrows_out, agg_out, deg_out,
             src_v, dst_v, ebuf0_v, ebuf1_v, hist_v,
             agg_sh, sem0, sem1):
  c = lax.axis_index("c")
  s = lax.axis_index("s")
  wid = s * NC + c
  ones16 = jnp.ones((L,), jnp.float32)
  ebufs = (ebuf0_v, ebuf1_v)
  sems = (sem0, sem1)

  pltpu.sync_copy(z1_hbm, hist_v)
  pltpu.sync_copy(za_hbm, agg_sh.at[pl.ds(s * SC_ROWS, SC_ROWS)])

  for j in range(PR_T // CH):
    pltpu.sync_copy(prev_hbm.at[pl.ds(wid * PR_T + j * CH, CH)],
                    src_v.at[0])
    pltpu.async_copy(h_hbm.at[src_v.at[0]], ebuf0_v, sem0).wait()
    pltpu.sync_copy(ebuf0_v, prevrows_out.at[pl.ds(wid * PR_T + j * CH, CH)])
  plsc.subcore_barrier()

  def sup(g, carry):
    pltpu.sync_copy(src_hbm.at[pl.ds(wid * NCH + g * SUP, SUP)], src_v)
    pltpu.sync_copy(dst_hbm.at[pl.ds(wid * NCH + g * SUP, SUP)], dst_v)
    descs = [None] * SUP
    descs[0] = pltpu.async_copy(h_hbm.at[src_v.at[0]], ebuf0_v, sem0)
    for j in range(SUP):
      if j + 1 < SUP:
        descs[j + 1] = pltpu.async_copy(
            h_hbm.at[src_v.at[j + 1]], ebufs[(j + 1) % 2], sems[(j + 1) % 2])
      descs[j].wait()
      for k in range(CH // L):
        plsc.addupdate_scatter(hist_v, [dst_v[j, pl.ds(k * L, L)]], ones16)
      pltpu.sync_copy(ebufs[j % 2], agg_sh.at[dst_v.at[j]], add=True)
    return carry

  lax.fori_loop(0, NSUP, sup, 0)
  plsc.subcore_barrier()

  rows = pl.ds(s * SC_ROWS, SC_ROWS)
  pltpu.sync_copy(agg_sh.at[rows], agg_out.at[c, rows])
  pltpu.sync_copy(hist_v, deg_out.at[pl.ds(wid * PADN, PADN)])


_sc_layer = pl.kernel(
    _sc_body,
    out_type=(
        jax.ShapeDtypeStruct((PADP, D), jnp.float32),
        jax.ShapeDtypeStruct((NC, PADN, D), jnp.float32),
        jax.ShapeDtypeStruct((NW * PADN,), jnp.float32),
    ),
    mesh=_mesh,
    compiler_params=pltpu.CompilerParams(needs_layout_passes=False),
    scratch_types=[
        pltpu.VMEM((SUP, CH), jnp.int32),
        pltpu.VMEM((SUP, CH), jnp.int32),
        pltpu.VMEM((CH, D), jnp.float32),
        pltpu.VMEM((CH, D), jnp.float32),
        pltpu.VMEM((PADN,), jnp.float32),
        pltpu.VMEM_SHARED((PADN, D), jnp.float32),
        pltpu.SemaphoreType.DMA,
        pltpu.SemaphoreType.DMA,
    ],
)

BR = 632


def _row_mask(i):
  gid = i * BR + lax.broadcasted_iota(jnp.int32, (BR, 1), 0)
  return gid < N


def _mm_body(x_ref, w_ref, b_ref, o_ref):
  out = jnp.dot(x_ref[...], w_ref[...],
                preferred_element_type=jnp.float32,
                precision=lax.Precision.HIGHEST) + b_ref[...]
  o_ref[...] = jnp.where(_row_mask(pl.program_id(0)), out, 0.0)


def _tc_matmul(x, w, b):
  return pl.pallas_call(
      _mm_body,
      grid=(PADN // BR,),
      in_specs=[
          pl.BlockSpec((BR, D), lambda i: (i, 0)),
          pl.BlockSpec((D, D), lambda i: (0, 0)),
          pl.BlockSpec((1, D), lambda i: (0, 0)),
      ],
      out_specs=pl.BlockSpec((BR, D), lambda i: (i, 0)),
      out_shape=jax.ShapeDtypeStruct((PADN, D), jnp.float32),
  )(x, w, b)


def _comb_body(prev_ref, aggA_ref, aggB_ref, dh_ref,
               wt_ref, wb_ref, b_ref, o_ref):
  deg = jnp.maximum(jnp.sum(dh_ref[...], axis=1, keepdims=True), 1.0)
  hn = jnp.maximum((aggA_ref[...] + aggB_ref[...]) / deg, 0.0)
  hp = jnp.maximum(prev_ref[...], 0.0)
  out = (
      jnp.dot(hp, wt_ref[...], preferred_element_type=jnp.float32,
              precision=lax.Precision.HIGHEST)
      + jnp.dot(hn, wb_ref[...], preferred_element_type=jnp.float32,
                precision=lax.Precision.HIGHEST)
      + b_ref[...])
  o_ref[...] = jnp.where(_row_mask(pl.program_id(0)), out, 0.0)


def _tc_combine(prevrows, agg, dh, wt, wb, b):
  return pl.pallas_call(
      _comb_body,
      grid=(PADN // BR,),
      in_specs=[
          pl.BlockSpec((BR, D), lambda i: (i, 0)),
          pl.BlockSpec((BR, D), lambda i: (i, 0)),
          pl.BlockSpec((BR, D), lambda i: (i, 0)),
          pl.BlockSpec((BR, NW), lambda i: (i, 0)),
          pl.BlockSpec((D, D), lambda i: (0, 0)),
          pl.BlockSpec((D, D), lambda i: (0, 0)),
          pl.BlockSpec((1, D), lambda i: (0, 0)),
      ],
      out_specs=pl.BlockSpec((BR, D), lambda i: (i, 0)),
      out_shape=jax.ShapeDtypeStruct((PADN, D), jnp.float32),
  )(prevrows, agg[0], agg[1], dh, wt, wb, b)


def kernel(x, adjs, previous_indices, W1, b1, W2, b2, Wo, bo):
  f32 = jnp.float32
  xp = jnp.zeros((PADN, D), f32).at[:N].set(x)
  za = jnp.zeros((SC_ROWS, D), f32)
  z1 = jnp.zeros((PADN,), f32)

  pad_dst = (N + (jnp.arange(EPAD - E) % (PADN - N))).astype(jnp.int32)
  pad_src = pad_dst
  pad_prev = jnp.zeros((PADP - N,), jnp.int32)

  def prep(layer):
    src = jnp.concatenate([adjs[layer, 1].astype(jnp.int32), pad_src])
    dst = jnp.concatenate([adjs[layer, 0].astype(jnp.int32), pad_dst])
    prev = jnp.concatenate(
        [previous_indices[layer].astype(jnp.int32), pad_prev])
    return src.reshape(-1, CH), dst.reshape(-1, CH), prev

  h1 = _tc_matmul(xp, W1, b1.reshape(1, D))

  src0, dst0, prev0 = prep(0)
  pr1, agg1, deg1 = _sc_layer(h1, src0, dst0, prev0, za, z1)
  dh1 = deg1.reshape(NW, PADN).T
  h2 = _tc_combine(pr1, agg1, dh1, W2[:NHID], W2[NHID:], b2.reshape(1, D))

  src1, dst1, prev1 = prep(1)
  pr2, agg2, deg2 = _sc_layer(h2, src1, dst1, prev1, za, z1)
  dh2 = deg2.reshape(NW, PADN).T
  wo_p = jnp.zeros((2 * NHID, D), f32).at[:, :NCLS].set(Wo)
  bo_p = jnp.zeros((1, D), f32).at[0, :NCLS].set(bo)
  outp = _tc_combine(pr2, agg2, dh2, wo_p[:NHID], wo_p[NHID:], bo_p)

  return outp[:N, :NCLS]


# speedup vs baseline: 6.2834x; 1.0013x over previous
import jax
import jax.numpy as jnp
from jax import lax
from jax.experimental import pallas as pl
from jax.experimental.pallas import tpu as pltpu
from jax.experimental.pallas import tpu_sc as plsc

N = 10000
D = 128
NHID = 128
NCLS = 40
E = 320000

NC, NS, L = 2, 16, 16
NW = NC * NS
PADN = 10112
CH = 128
EPAD = 327680
NCH = EPAD // NW // CH
SUP = 8
NSUP = NCH // SUP
PADP = 12288
PR_T = PADP // NW
SC_ROWS = PADN // NS

_mesh = plsc.VectorSubcoreMesh(
    core_axis_name="c", subcore_axis_name="s", num_cores=NC, num_subcores=NS)


def _sc_body(h_hbm, src_hbm, dst_hbm, prev_hbm, za_hbm, z1_hbm,
             prevrows_out, agg_out, deg_out,
             src_v, dst_v, ebuf0_v, ebuf1_v, hist_v,
             agg_sh, sem0, sem1, ssem0, ssem1):
  c = lax.axis_index("c")
  s = lax.axis_index("s")
  wid = s * NC + c
  ones16 = jnp.ones((L,), jnp.float32)
  ebufs = (ebuf0_v, ebuf1_v)
  sems = (sem0, sem1)
  ssems = (ssem0, ssem1)

  pltpu.sync_copy(z1_hbm, hist_v)
  pltpu.sync_copy(za_hbm, agg_sh.at[pl.ds(s * SC_ROWS, SC_ROWS)])

  for j in range(PR_T // CH):
    pltpu.sync_copy(prev_hbm.at[pl.ds(wid * PR_T + j * CH, CH)],
                    src_v.at[0])
    pltpu.async_copy(h_hbm.at[src_v.at[0]], ebuf0_v, sem0).wait()
    pltpu.sync_copy(ebuf0_v, prevrows_out.at[pl.ds(wid * PR_T + j * CH, CH)])
  plsc.subcore_barrier()

  def sup(g, carry):
    pltpu.sync_copy(src_hbm.at[pl.ds(wid * NCH + g * SUP, SUP)], src_v)
    pltpu.sync_copy(dst_hbm.at[pl.ds(wid * NCH + g * SUP, SUP)], dst_v)
    gd = [None] * SUP
    sd = [None] * SUP
    gd[0] = pltpu.async_copy(h_hbm.at[src_v.at[0]], ebuf0_v, sem0)
    for j in range(SUP):
      if j >= 1:
        sd[j - 1].wait()
      if j + 1 < SUP:
        gd[j + 1] = pltpu.async_copy(
            h_hbm.at[src_v.at[j + 1]], ebufs[(j + 1) % 2], sems[(j + 1) % 2])
      gd[j].wait()
      for k in range(CH // L):
        plsc.addupdate_scatter(hist_v, [dst_v[j, pl.ds(k * L, L)]], ones16)
      sd[j] = pltpu.async_copy(ebufs[j % 2], agg_sh.at[dst_v.at[j]],
                               ssems[j % 2], add=True)
    sd[SUP - 1].wait()
    return carry

  lax.fori_loop(0, NSUP, sup, 0)
  plsc.subcore_barrier()

  rows = pl.ds(s * SC_ROWS, SC_ROWS)
  pltpu.sync_copy(agg_sh.at[rows], agg_out.at[c, rows])
  pltpu.sync_copy(hist_v, deg_out.at[pl.ds(wid * PADN, PADN)])


_sc_layer = pl.kernel(
    _sc_body,
    out_type=(
        jax.ShapeDtypeStruct((PADP, D), jnp.float32),
        jax.ShapeDtypeStruct((NC, PADN, D), jnp.float32),
        jax.ShapeDtypeStruct((NW * PADN,), jnp.float32),
    ),
    mesh=_mesh,
    compiler_params=pltpu.CompilerParams(needs_layout_passes=False),
    scratch_types=[
        pltpu.VMEM((SUP, CH), jnp.int32),
        pltpu.VMEM((SUP, CH), jnp.int32),
        pltpu.VMEM((CH, D), jnp.float32),
        pltpu.VMEM((CH, D), jnp.float32),
        pltpu.VMEM((PADN,), jnp.float32),
        pltpu.VMEM_SHARED((PADN, D), jnp.float32),
        pltpu.SemaphoreType.DMA,
        pltpu.SemaphoreType.DMA,
        pltpu.SemaphoreType.DMA,
        pltpu.SemaphoreType.DMA,
    ],
)

BR = 632


def _row_mask(i):
  gid = i * BR + lax.broadcasted_iota(jnp.int32, (BR, 1), 0)
  return gid < N


def _mm_body(x_ref, w_ref, b_ref, o_ref):
  out = jnp.dot(x_ref[...], w_ref[...],
                preferred_element_type=jnp.float32,
                precision=lax.Precision.HIGHEST) + b_ref[...]
  o_ref[...] = jnp.where(_row_mask(pl.program_id(0)), out, 0.0)


def _tc_matmul(x, w, b):
  return pl.pallas_call(
      _mm_body,
      grid=(PADN // BR,),
      in_specs=[
          pl.BlockSpec((BR, D), lambda i: (i, 0)),
          pl.BlockSpec((D, D), lambda i: (0, 0)),
          pl.BlockSpec((1, D), lambda i: (0, 0)),
      ],
      out_specs=pl.BlockSpec((BR, D), lambda i: (i, 0)),
      out_shape=jax.ShapeDtypeStruct((PADN, D), jnp.float32),
  )(x, w, b)


def _comb_body(prev_ref, aggA_ref, aggB_ref, dh_ref,
               wt_ref, wb_ref, b_ref, o_ref):
  deg = jnp.maximum(jnp.sum(dh_ref[...], axis=1, keepdims=True), 1.0)
  hn = jnp.maximum((aggA_ref[...] + aggB_ref[...]) / deg, 0.0)
  hp = jnp.maximum(prev_ref[...], 0.0)
  out = (
      jnp.dot(hp, wt_ref[...], preferred_element_type=jnp.float32,
              precision=lax.Precision.HIGHEST)
      + jnp.dot(hn, wb_ref[...], preferred_element_type=jnp.float32,
                precision=lax.Precision.HIGHEST)
      + b_ref[...])
  o_ref[...] = jnp.where(_row_mask(pl.program_id(0)), out, 0.0)


def _tc_combine(prevrows, agg, dh, wt, wb, b):
  return pl.pallas_call(
      _comb_body,
      grid=(PADN // BR,),
      in_specs=[
          pl.BlockSpec((BR, D), lambda i: (i, 0)),
          pl.BlockSpec((BR, D), lambda i: (i, 0)),
          pl.BlockSpec((BR, D), lambda i: (i, 0)),
          pl.BlockSpec((BR, NW), lambda i: (i, 0)),
          pl.BlockSpec((D, D), lambda i: (0, 0)),
          pl.BlockSpec((D, D), lambda i: (0, 0)),
          pl.BlockSpec((1, D), lambda i: (0, 0)),
      ],
      out_specs=pl.BlockSpec((BR, D), lambda i: (i, 0)),
      out_shape=jax.ShapeDtypeStruct((PADN, D), jnp.float32),
  )(prevrows, agg[0], agg[1], dh, wt, wb, b)


def kernel(x, adjs, previous_indices, W1, b1, W2, b2, Wo, bo):
  f32 = jnp.float32
  xp = jnp.zeros((PADN, D), f32).at[:N].set(x)
  za = jnp.zeros((SC_ROWS, D), f32)
  z1 = jnp.zeros((PADN,), f32)

  pad_dst = (N + (jnp.arange(EPAD - E) % (PADN - N))).astype(jnp.int32)
  pad_src = pad_dst
  pad_prev = jnp.zeros((PADP - N,), jnp.int32)

  def prep(layer):
    src = jnp.concatenate([adjs[layer, 1].astype(jnp.int32), pad_src])
    dst = jnp.concatenate([adjs[layer, 0].astype(jnp.int32), pad_dst])
    prev = jnp.concatenate(
        [previous_indices[layer].astype(jnp.int32), pad_prev])
    return src.reshape(-1, CH), dst.reshape(-1, CH), prev

  h1 = _tc_matmul(xp, W1, b1.reshape(1, D))

  src0, dst0, prev0 = prep(0)
  pr1, agg1, deg1 = _sc_layer(h1, src0, dst0, prev0, za, z1)
  dh1 = deg1.reshape(NW, PADN).T
  h2 = _tc_combine(pr1, agg1, dh1, W2[:NHID], W2[NHID:], b2.reshape(1, D))

  src1, dst1, prev1 = prep(1)
  pr2, agg2, deg2 = _sc_layer(h2, src1, dst1, prev1, za, z1)
  dh2 = deg2.reshape(NW, PADN).T
  wo_p = jnp.zeros((2 * NHID, D), f32).at[:, :NCLS].set(Wo)
  bo_p = jnp.zeros((1, D), f32).at[0, :NCLS].set(bo)
  outp = _tc_combine(pr2, agg2, dh2, wo_p[:NHID], wo_p[NHID:], bo_p)

  return outp[:N, :NCLS]


# speedup vs baseline: 6.8574x; 1.0914x over previous
import jax
import jax.numpy as jnp
from jax import lax
from jax.experimental import pallas as pl
from jax.experimental.pallas import tpu as pltpu
from jax.experimental.pallas import tpu_sc as plsc

N = 10000
D = 128
NHID = 128
NCLS = 40
E = 320000

NC, NS, L = 2, 16, 16
NW = NC * NS
PADN = 10112
CH = 128
EPAD = 327680
NCH = EPAD // NW // CH
SUP = 4
NSUP = NCH // SUP
TP = NSUP // 2
PADP = 12288
PR_T = PADP // NW
SC_ROWS = PADN // NS

_mesh = plsc.VectorSubcoreMesh(
    core_axis_name="c", subcore_axis_name="s", num_cores=NC, num_subcores=NS)


def _sc_body(h_hbm, src_hbm, dst_hbm, prev_hbm, za_hbm, z1_hbm,
             prevrows_out, agg_out, deg_out,
             src0_v, dst0_v, src1_v, dst1_v, ebuf0_v, ebuf1_v, hist_v,
             agg_sh, sem0, sem1, ssem0, ssem1, isem0, isem1):
  c = lax.axis_index("c")
  s = lax.axis_index("s")
  wid = s * NC + c
  ones16 = jnp.ones((L,), jnp.float32)
  ebufs = (ebuf0_v, ebuf1_v)
  sems = (sem0, sem1)
  ssems = (ssem0, ssem1)
  slots = ((src0_v, dst0_v, isem0), (src1_v, dst1_v, isem1))

  pltpu.sync_copy(z1_hbm, hist_v)
  pltpu.sync_copy(za_hbm, agg_sh.at[pl.ds(s * SC_ROWS, SC_ROWS)])

  for j in range(PR_T // CH):
    pltpu.sync_copy(prev_hbm.at[pl.ds(wid * PR_T + j * CH, CH)],
                    src0_v.at[0])
    pltpu.async_copy(h_hbm.at[src0_v.at[0]], ebuf0_v, sem0).wait()
    pltpu.sync_copy(ebuf0_v, prevrows_out.at[pl.ds(wid * PR_T + j * CH, CH)])
  plsc.subcore_barrier()

  def prefetch(slot, sup_idx):
    sv, dv, isem = slots[slot]
    d1 = pltpu.async_copy(src_hbm.at[pl.ds(wid * NCH + sup_idx * SUP, SUP)],
                          sv, isem)
    d2 = pltpu.async_copy(dst_hbm.at[pl.ds(wid * NCH + sup_idx * SUP, SUP)],
                          dv, isem)
    return d1, d2

  def wait_prefetch(slot):
    sv, dv, isem = slots[slot]
    pltpu.make_async_copy(src_hbm.at[pl.ds(0, SUP)], sv, isem).wait()
    pltpu.make_async_copy(dst_hbm.at[pl.ds(0, SUP)], dv, isem).wait()

  d1, d2 = prefetch(0, 0)
  d1.wait()
  d2.wait()
  prefetch(1, 1)
  pltpu.async_copy(h_hbm.at[src0_v.at[0]], ebuf0_v, sem0)

  def pair(t, carry):
    gd = [None] * 8
    sd = [None] * 8
    nxt0 = 2 * t + 2
    nxt1 = 2 * t + 3
    for G in range(8):
      half, j = G // 4, G % 4
      sv, dv, _ = slots[half]
      if G == 3:
        wait_prefetch(1)
      if G == 5:
        @pl.when(t + 1 < TP)
        def _():
          prefetch(0, nxt0)
      if G >= 1:
        sd[G - 1].wait()
      if G + 1 < 8:
        nh, nj = (G + 1) // 4, (G + 1) % 4
        nsv = slots[nh][0]
        gd[G + 1] = pltpu.async_copy(
            h_hbm.at[nsv.at[nj]], ebufs[(G + 1) % 2], sems[(G + 1) % 2])
      else:
        @pl.when(t + 1 < TP)
        def _():
          wait_prefetch(0)
          pltpu.async_copy(h_hbm.at[src0_v.at[0]], ebuf0_v, sem0)
      if gd[G] is None:
        pltpu.make_async_copy(h_hbm.at[src0_v.at[0]], ebuf0_v, sem0).wait()
      else:
        gd[G].wait()
      for k in range(CH // L):
        plsc.addupdate_scatter(hist_v, [dv[j, pl.ds(k * L, L)]], ones16)
      sd[G] = pltpu.async_copy(ebufs[G % 2], agg_sh.at[dv.at[j]],
                               ssems[G % 2], add=True)
    sd[7].wait()

    @pl.when(t + 1 < TP)
    def _():
      prefetch(1, nxt1)

    return carry

  lax.fori_loop(0, TP, pair, 0)
  plsc.subcore_barrier()

  rows = pl.ds(s * SC_ROWS, SC_ROWS)
  pltpu.sync_copy(agg_sh.at[rows], agg_out.at[c, rows])
  pltpu.sync_copy(hist_v, deg_out.at[pl.ds(wid * PADN, PADN)])


_sc_layer = pl.kernel(
    _sc_body,
    out_type=(
        jax.ShapeDtypeStruct((PADP, D), jnp.float32),
        jax.ShapeDtypeStruct((NC, PADN, D), jnp.float32),
        jax.ShapeDtypeStruct((NW * PADN,), jnp.float32),
    ),
    mesh=_mesh,
    compiler_params=pltpu.CompilerParams(needs_layout_passes=False),
    scratch_types=[
        pltpu.VMEM((SUP, CH), jnp.int32),
        pltpu.VMEM((SUP, CH), jnp.int32),
        pltpu.VMEM((SUP, CH), jnp.int32),
        pltpu.VMEM((SUP, CH), jnp.int32),
        pltpu.VMEM((CH, D), jnp.float32),
        pltpu.VMEM((CH, D), jnp.float32),
        pltpu.VMEM((PADN,), jnp.float32),
        pltpu.VMEM_SHARED((PADN, D), jnp.float32),
        pltpu.SemaphoreType.DMA,
        pltpu.SemaphoreType.DMA,
        pltpu.SemaphoreType.DMA,
        pltpu.SemaphoreType.DMA,
        pltpu.SemaphoreType.DMA,
        pltpu.SemaphoreType.DMA,
    ],
)

BR = 632


def _row_mask(i):
  gid = i * BR + lax.broadcasted_iota(jnp.int32, (BR, 1), 0)
  return gid < N


def _mm_body(x_ref, w_ref, b_ref, o_ref):
  out = jnp.dot(x_ref[...], w_ref[...],
                preferred_element_type=jnp.float32,
                precision=lax.Precision.HIGHEST) + b_ref[...]
  o_ref[...] = jnp.where(_row_mask(pl.program_id(0)), out, 0.0)


def _tc_matmul(x, w, b):
  return pl.pallas_call(
      _mm_body,
      grid=(PADN // BR,),
      in_specs=[
          pl.BlockSpec((BR, D), lambda i: (i, 0)),
          pl.BlockSpec((D, D), lambda i: (0, 0)),
          pl.BlockSpec((1, D), lambda i: (0, 0)),
      ],
      out_specs=pl.BlockSpec((BR, D), lambda i: (i, 0)),
      out_shape=jax.ShapeDtypeStruct((PADN, D), jnp.float32),
  )(x, w, b)


def _comb_body(prev_ref, aggA_ref, aggB_ref, dh_ref,
               wt_ref, wb_ref, b_ref, o_ref):
  deg = jnp.maximum(jnp.sum(dh_ref[...], axis=1, keepdims=True), 1.0)
  hn = jnp.maximum((aggA_ref[...] + aggB_ref[...]) / deg, 0.0)
  hp = jnp.maximum(prev_ref[...], 0.0)
  out = (
      jnp.dot(hp, wt_ref[...], preferred_element_type=jnp.float32,
              precision=lax.Precision.HIGHEST)
      + jnp.dot(hn, wb_ref[...], preferred_element_type=jnp.float32,
                precision=lax.Precision.HIGHEST)
      + b_ref[...])
  o_ref[...] = jnp.where(_row_mask(pl.program_id(0)), out, 0.0)


def _tc_combine(prevrows, agg, dh, wt, wb, b):
  return pl.pallas_call(
      _comb_body,
      grid=(PADN // BR,),
      in_specs=[
          pl.BlockSpec((BR, D), lambda i: (i, 0)),
          pl.BlockSpec((BR, D), lambda i: (i, 0)),
          pl.BlockSpec((BR, D), lambda i: (i, 0)),
          pl.BlockSpec((BR, NW), lambda i: (i, 0)),
          pl.BlockSpec((D, D), lambda i: (0, 0)),
          pl.BlockSpec((D, D), lambda i: (0, 0)),
          pl.BlockSpec((1, D), lambda i: (0, 0)),
      ],
      out_specs=pl.BlockSpec((BR, D), lambda i: (i, 0)),
      out_shape=jax.ShapeDtypeStruct((PADN, D), jnp.float32),
  )(prevrows, agg[0], agg[1], dh, wt, wb, b)


def kernel(x, adjs, previous_indices, W1, b1, W2, b2, Wo, bo):
  f32 = jnp.float32
  xp = jnp.zeros((PADN, D), f32).at[:N].set(x)
  za = jnp.zeros((SC_ROWS, D), f32)
  z1 = jnp.zeros((PADN,), f32)

  pad_dst = (N + (jnp.arange(EPAD - E) % (PADN - N))).astype(jnp.int32)
  pad_src = pad_dst
  pad_prev = jnp.zeros((PADP - N,), jnp.int32)

  def prep(layer):
    src = jnp.concatenate([adjs[layer, 1].astype(jnp.int32), pad_src])
    dst = jnp.concatenate([adjs[layer, 0].astype(jnp.int32), pad_dst])
    prev = jnp.concatenate(
        [previous_indices[layer].astype(jnp.int32), pad_prev])
    return src.reshape(-1, CH), dst.reshape(-1, CH), prev

  h1 = _tc_matmul(xp, W1, b1.reshape(1, D))

  src0, dst0, prev0 = prep(0)
  pr1, agg1, deg1 = _sc_layer(h1, src0, dst0, prev0, za, z1)
  dh1 = deg1.reshape(NW, PADN).T
  h2 = _tc_combine(pr1, agg1, dh1, W2[:NHID], W2[NHID:], b2.reshape(1, D))

  src1, dst1, prev1 = prep(1)
  pr2, agg2, deg2 = _sc_layer(h2, src1, dst1, prev1, za, z1)
  dh2 = deg2.reshape(NW, PADN).T
  wo_p = jnp.zeros((2 * NHID, D), f32).at[:, :NCLS].set(Wo)
  bo_p = jnp.zeros((1, D), f32).at[0, :NCLS].set(bo)
  outp = _tc_combine(pr2, agg2, dh2, wo_p[:NHID], wo_p[NHID:], bo_p)

  return outp[:N, :NCLS]
